# Initial kernel scaffold; baseline (speedup 1.0000x reference)
import jax
import jax.numpy as jnp
from jax.experimental import pallas as pl


def kernel(emb, adj, W1, b1, W2, b2):
    raise NotImplementedError("write your pallas kernel here")


# speedup vs baseline: 5429.4468x; 5429.4468x over previous
import jax
import jax.numpy as jnp
from jax.experimental import pallas as pl


def _fused_gcn(emb_ref, adj_ref, w1_ref, b1_ref, w2_ref, b2_ref, out_ref):
    adj = adj_ref[...]
    n = adj.shape[0]

    ones_col = jnp.ones((n, 1), dtype=jnp.float32)
    deg = jax.lax.dot_general(
        adj, ones_col, (((0,), (0,)), ((), ())),
        preferred_element_type=jnp.float32) + 1.0
    dinv = jax.lax.rsqrt(deg)

    s1 = jnp.dot(emb_ref[...], w1_ref[...],
                 preferred_element_type=jnp.float32) * dinv
    t1 = jax.lax.dot_general(
        adj, s1, (((0,), (0,)), ((), ())),
        preferred_element_type=jnp.float32) + s1
    h1 = jnp.maximum(t1 * dinv + b1_ref[...], 0.0)

    s2 = jnp.dot(h1, w2_ref[...], preferred_element_type=jnp.float32) * dinv
    t2 = jax.lax.dot_general(
        adj, s2, (((0,), (0,)), ((), ())),
        preferred_element_type=jnp.float32) + s2
    out_ref[...] = t2 * dinv + b2_ref[...]


def kernel(emb, adj, W1, b1, W2, b2):
    n = emb.shape[0]
    out_dim = W2.shape[1]
    return pl.pallas_call(
        _fused_gcn,
        out_shape=jax.ShapeDtypeStruct((n, out_dim), jnp.float32),
    )(emb, adj, W1, b1.reshape(1, -1), W2, b2.reshape(1, -1))
